# baseline (device time: 12248 ns/iter reference)
import jax
import jax.numpy as jnp
from jax import lax
from jax.experimental import pallas as pl
from jax.experimental.pallas import tpu as pltpu

M = 512
HALF = 256
K = 4
CH = HALF // K


def kernel(x):
    m, n = x.shape
    bf16 = jnp.bfloat16

    def body(x_ref, out_ref, send_buf,
             x_send_sems, x_recv_sems, y_send_sems, y_recv_sems):
        my_x = lax.axis_index("x")
        my_y = lax.axis_index("y")
        px = 1 - my_x
        py = 1 - my_y

        barrier_sem = pltpu.get_barrier_semaphore()
        pl.semaphore_signal(
            barrier_sem, inc=1,
            device_id=(px, my_y), device_id_type=pl.DeviceIdType.MESH,
        )
        pl.semaphore_signal(
            barrier_sem, inc=1,
            device_id=(my_x, py), device_id_type=pl.DeviceIdType.MESH,
        )

        def pack(c):
            send_buf[pl.ds(c * CH, CH), :] = x_ref[
                pl.ds(my_y * HALF + c * CH, CH), pl.ds(px * M, M)
            ].astype(bf16)

        pack(0)
        pl.semaphore_wait(barrier_sem, 2)

        x_rdmas = []
        for c in range(K):
            r = pltpu.make_async_remote_copy(
                src_ref=send_buf.at[pl.ds(c * CH, CH), :],
                dst_ref=out_ref.at[pl.ds(my_x * M + my_y * HALF + c * CH, CH), :],
                send_sem=x_send_sems.at[c],
                recv_sem=x_recv_sems.at[c],
                device_id=(px, my_y),
                device_id_type=pl.DeviceIdType.MESH,
            )
            r.start()
            x_rdmas.append(r)
            if c + 1 < K:
                pack(c + 1)

        out_ref[pl.ds(my_x * M, M), :] = x_ref[:, pl.ds(my_x * M, M)].astype(bf16)

        y_fwd = []
        for c in range(K):
            x_rdmas[c].wait_recv()
            rows = pl.ds(px * M + my_y * HALF + c * CH, CH)
            f = pltpu.make_async_remote_copy(
                src_ref=out_ref.at[rows, :],
                dst_ref=out_ref.at[rows, :],
                send_sem=y_send_sems.at[c],
                recv_sem=y_recv_sems.at[c],
                device_id=(my_x, py),
                device_id_type=pl.DeviceIdType.MESH,
            )
            f.start()
            y_fwd.append(f)

        for c in range(K):
            y_fwd[c].wait_recv()
            x_rdmas[c].wait_send()
            y_fwd[c].wait_send()

    return pl.pallas_call(
        body,
        out_shape=jax.ShapeDtypeStruct((2 * m, M), jnp.bfloat16),
        in_specs=[pl.BlockSpec(memory_space=pltpu.VMEM)],
        out_specs=pl.BlockSpec(memory_space=pltpu.VMEM),
        scratch_shapes=[
            pltpu.VMEM((HALF, M), jnp.bfloat16),
            pltpu.SemaphoreType.DMA((K,)),
            pltpu.SemaphoreType.DMA((K,)),
            pltpu.SemaphoreType.DMA((K,)),
            pltpu.SemaphoreType.DMA((K,)),
        ],
        compiler_params=pltpu.CompilerParams(collective_id=0),
    )(x)


# device time: 10624 ns/iter; 1.1529x vs baseline; 1.1529x over previous
import jax
import jax.numpy as jnp
from jax import lax
from jax.experimental import pallas as pl
from jax.experimental.pallas import tpu as pltpu

M = 512
FWD = 128
REST = M - 2 * FWD
K = 4
CH = FWD // K


def kernel(x):
    m, n = x.shape
    bf16 = jnp.bfloat16

    def body(x_ref, out_ref, send_buf,
             x_send_sems, x_recv_sems, y_send_sems, y_recv_sems, y_ready):
        my_x = lax.axis_index("x")
        my_y = lax.axis_index("y")
        px = 1 - my_x
        py = 1 - my_y

        f0 = my_y * (M - FWD)

        barrier_sem = pltpu.get_barrier_semaphore()
        pl.semaphore_signal(
            barrier_sem, inc=1,
            device_id=(px, my_y), device_id_type=pl.DeviceIdType.MESH,
        )
        pl.semaphore_signal(
            y_ready, inc=1,
            device_id=(my_x, py), device_id_type=pl.DeviceIdType.MESH,
        )

        send_buf[pl.ds(f0, FWD), :] = x_ref[
            pl.ds(f0, FWD), pl.ds(px * M, M)
        ].astype(bf16)
        send_buf[pl.ds(FWD, REST), :] = x_ref[
            pl.ds(FWD, REST), pl.ds(px * M, M)
        ].astype(bf16)

        pl.semaphore_wait(barrier_sem, 1)

        x_rdmas = []
        for c in range(K):
            r = pltpu.make_async_remote_copy(
                src_ref=send_buf.at[pl.ds(f0 + c * CH, CH), :],
                dst_ref=out_ref.at[pl.ds(my_x * M + f0 + c * CH, CH), :],
                send_sem=x_send_sems.at[c],
                recv_sem=x_recv_sems.at[c],
                device_id=(px, my_y),
                device_id_type=pl.DeviceIdType.MESH,
            )
            r.start()
            x_rdmas.append(r)

        rest = pltpu.make_async_remote_copy(
            src_ref=send_buf.at[pl.ds(FWD, REST), :],
            dst_ref=out_ref.at[pl.ds(my_x * M + FWD, REST), :],
            send_sem=x_send_sems.at[K],
            recv_sem=x_recv_sems.at[K],
            device_id=(px, my_y),
            device_id_type=pl.DeviceIdType.MESH,
        )
        rest.start()

        out_ref[pl.ds(my_x * M, M), :] = x_ref[:, pl.ds(my_x * M, M)].astype(bf16)

        y_fwd = []
        for c in range(K):
            x_rdmas[c].wait_recv()
            if c == 0:
                pl.semaphore_wait(y_ready, 1)
            rows = pl.ds(px * M + f0 + c * CH, CH)
            f = pltpu.make_async_remote_copy(
                src_ref=out_ref.at[rows, :],
                dst_ref=out_ref.at[rows, :],
                send_sem=y_send_sems.at[c],
                recv_sem=y_recv_sems.at[c],
                device_id=(my_x, py),
                device_id_type=pl.DeviceIdType.MESH,
            )
            f.start()
            y_fwd.append(f)

        rest.wait_recv()

        for c in range(K):
            y_fwd[c].wait_recv()
            x_rdmas[c].wait_send()
            y_fwd[c].wait_send()
        rest.wait_send()

    return pl.pallas_call(
        body,
        out_shape=jax.ShapeDtypeStruct((2 * m, M), jnp.bfloat16),
        in_specs=[pl.BlockSpec(memory_space=pltpu.VMEM)],
        out_specs=pl.BlockSpec(memory_space=pltpu.VMEM),
        scratch_shapes=[
            pltpu.VMEM((M, M), jnp.bfloat16),
            pltpu.SemaphoreType.DMA((K + 1,)),
            pltpu.SemaphoreType.DMA((K + 1,)),
            pltpu.SemaphoreType.DMA((K,)),
            pltpu.SemaphoreType.DMA((K,)),
            pltpu.SemaphoreType.REGULAR,
        ],
        compiler_params=pltpu.CompilerParams(collective_id=0),
    )(x)
